# Initial kernel scaffold; baseline (speedup 1.0000x reference)
import jax
import jax.numpy as jnp
from jax.experimental import pallas as pl


def kernel(items_emb):
    raise NotImplementedError("write your pallas kernel here")


# speedup vs baseline: 3.1237x; 3.1237x over previous
import functools

import jax
import jax.numpy as jnp
import numpy as np
from jax.experimental import pallas as pl
from jax.experimental.pallas import tpu as pltpu

_TARGET_ITEMS = (5, 17, 123, 999, 4242, 10000, 25000, 50000, 75000, 99999)
_K = 10
_ALPHA = 1.0
_ITEMS_LIMIT = 60

_N_ROWS = 100000
_DIM = 64
_BLOCK_ROWS = 512
_N_BLOCKS = (_N_ROWS + _BLOCK_ROWS - 1) // _BLOCK_ROWS


def _attack_kernel(x_ref, emb_any, out_ref, norms_s, row_s, acc_s, sem):
    i = pl.program_id(0)

    x = x_ref[...]
    n2 = jnp.sum(x * x, axis=1)
    rid = i * _BLOCK_ROWS + jax.lax.iota(jnp.int32, _BLOCK_ROWS)
    n = jnp.where(rid < _N_ROWS, jnp.sqrt(n2), -1.0)
    norms_s[i, :] = n

    @pl.when(i == _N_BLOCKS - 1)
    def _():
        out_ref[...] = jnp.zeros_like(out_ref)
        acc_s[...] = jnp.zeros_like(acc_s)

        fid = (
            jax.lax.broadcasted_iota(jnp.int32, (_N_BLOCKS, _BLOCK_ROWS), 0)
            * _BLOCK_ROWS
            + jax.lax.broadcasted_iota(jnp.int32, (_N_BLOCKS, _BLOCK_ROWS), 1)
        )
        nm = norms_s[...]
        for _ in range(_K):
            m = jnp.max(nm)
            idx = jnp.min(jnp.where(nm == m, fid, jnp.int32(2**31 - 1)))
            cp = pltpu.make_async_copy(
                emb_any.at[pl.ds(idx, 1), :], row_s, sem
            )
            cp.start()
            cp.wait()
            acc_s[...] += row_s[...]
            nm = jnp.where(fid == idx, -jnp.inf, nm)

        v = acc_s[...] / float(_K) * 10.0

        for j, t in enumerate(_TARGET_ITEMS):
            cp = pltpu.make_async_copy(
                emb_any.at[pl.ds(t, 1), :], row_s, sem
            )
            cp.start()
            cp.wait()
            nt = _ITEMS_LIMIT - 2 * len(_TARGET_ITEMS) + j
            out_ref[pl.ds(nt, 1), :] = (v - row_s[...]) * _ALPHA


@jax.jit
def kernel(items_emb):
    out_rows = 64
    upd = pl.pallas_call(
        _attack_kernel,
        grid=(_N_BLOCKS,),
        in_specs=[
            pl.BlockSpec((_BLOCK_ROWS, _DIM), lambda i: (i, 0)),
            pl.BlockSpec(memory_space=pl.ANY),
        ],
        out_specs=pl.BlockSpec((out_rows, _DIM), lambda i: (0, 0)),
        out_shape=jax.ShapeDtypeStruct((out_rows, _DIM), jnp.float32),
        scratch_shapes=[
            pltpu.VMEM((_N_BLOCKS, _BLOCK_ROWS), jnp.float32),
            pltpu.VMEM((1, _DIM), jnp.float32),
            pltpu.VMEM((1, _DIM), jnp.float32),
            pltpu.SemaphoreType.DMA,
        ],
        compiler_params=pltpu.CompilerParams(
            dimension_semantics=("arbitrary",),
        ),
    )(items_emb, items_emb)

    num_keep = _ITEMS_LIMIT - 2 * len(_TARGET_ITEMS)
    kept = [i for i in range(_N_ROWS) if i not in _TARGET_ITEMS][:num_keep]
    chosen_items = jnp.asarray(list(kept) + list(_TARGET_ITEMS), dtype=jnp.int32)
    return chosen_items, upd[: num_keep + len(_TARGET_ITEMS)]


# speedup vs baseline: 6.9353x; 2.2202x over previous
import functools

import jax
import jax.numpy as jnp
import numpy as np
from jax.experimental import pallas as pl
from jax.experimental.pallas import tpu as pltpu

_TARGET_ITEMS = (5, 17, 123, 999, 4242, 10000, 25000, 50000, 75000, 99999)
_K = 10
_ALPHA = 1.0
_ITEMS_LIMIT = 60
_NT = len(_TARGET_ITEMS)

_N_ROWS = 100000
_DIM = 64
_BLOCK_ROWS = 4096
_N_BLOCKS = (_N_ROWS + _BLOCK_ROWS - 1) // _BLOCK_ROWS


def _attack_kernel(
    x_ref, emb_any, out_ref, norms_s, tgt_rows, row_a, row_b, acc_s,
    tgt_sems, sem_a, sem_b
):
    i = pl.program_id(0)

    @pl.when(i == 0)
    def _():
        for j, t in enumerate(_TARGET_ITEMS):
            pltpu.make_async_copy(
                emb_any.at[pl.ds(t, 1), :],
                tgt_rows.at[pl.ds(j, 1), :],
                tgt_sems.at[j],
            ).start()

    x = x_ref[...]
    n2 = jnp.sum(x * x, axis=1)
    rid = i * _BLOCK_ROWS + jax.lax.iota(jnp.int32, _BLOCK_ROWS)
    norms_s[i, :] = jnp.where(rid < _N_ROWS, jnp.sqrt(n2), -1.0)

    @pl.when(i == _N_BLOCKS - 1)
    def _():
        out_ref[...] = jnp.zeros_like(out_ref)
        acc_s[...] = jnp.zeros_like(acc_s)

        fid = (
            jax.lax.broadcasted_iota(jnp.int32, (_N_BLOCKS, _BLOCK_ROWS), 0)
            * _BLOCK_ROWS
            + jax.lax.broadcasted_iota(jnp.int32, (_N_BLOCKS, _BLOCK_ROWS), 1)
        )
        nm = norms_s[...]

        bufs = (row_a, row_b)
        sems = (sem_a, sem_b)
        prev = None
        for k in range(_K):
            m = jnp.max(nm)
            idx = jnp.min(jnp.where(nm == m, fid, jnp.int32(2**31 - 1)))
            cp = pltpu.make_async_copy(
                emb_any.at[pl.ds(idx, 1), :], bufs[k % 2], sems[k % 2]
            )
            cp.start()
            if prev is not None:
                prev.wait()
                acc_s[...] += bufs[(k - 1) % 2][...]
            prev = cp
            nm = jnp.where(fid == idx, -jnp.inf, nm)
        prev.wait()
        acc_s[...] += bufs[(_K - 1) % 2][...]

        v = acc_s[...] / float(_K) * 10.0

        for j in range(_NT):
            pltpu.make_async_copy(
                emb_any.at[pl.ds(_TARGET_ITEMS[j], 1), :],
                tgt_rows.at[pl.ds(j, 1), :],
                tgt_sems.at[j],
            ).wait()
        nk = _ITEMS_LIMIT - 2 * _NT
        out_ref[pl.ds(nk, _NT), :] = (v - tgt_rows[...]) * _ALPHA


@jax.jit
def kernel(items_emb):
    out_rows = 64
    upd = pl.pallas_call(
        _attack_kernel,
        grid=(_N_BLOCKS,),
        in_specs=[
            pl.BlockSpec((_BLOCK_ROWS, _DIM), lambda i: (i, 0)),
            pl.BlockSpec(memory_space=pl.ANY),
        ],
        out_specs=pl.BlockSpec((out_rows, _DIM), lambda i: (0, 0)),
        out_shape=jax.ShapeDtypeStruct((out_rows, _DIM), jnp.float32),
        scratch_shapes=[
            pltpu.VMEM((_N_BLOCKS, _BLOCK_ROWS), jnp.float32),
            pltpu.VMEM((_NT, _DIM), jnp.float32),
            pltpu.VMEM((1, _DIM), jnp.float32),
            pltpu.VMEM((1, _DIM), jnp.float32),
            pltpu.VMEM((1, _DIM), jnp.float32),
            pltpu.SemaphoreType.DMA((_NT,)),
            pltpu.SemaphoreType.DMA,
            pltpu.SemaphoreType.DMA,
        ],
        compiler_params=pltpu.CompilerParams(
            dimension_semantics=("arbitrary",),
        ),
    )(items_emb, items_emb)

    num_keep = _ITEMS_LIMIT - 2 * _NT
    kept = [i for i in range(_N_ROWS) if i not in _TARGET_ITEMS][:num_keep]
    chosen_items = jnp.asarray(list(kept) + list(_TARGET_ITEMS), dtype=jnp.int32)
    return chosen_items, upd[: num_keep + _NT]


# speedup vs baseline: 7.5724x; 1.0919x over previous
import functools

import jax
import jax.numpy as jnp
import numpy as np
from jax.experimental import pallas as pl
from jax.experimental.pallas import tpu as pltpu

_TARGET_ITEMS = (5, 17, 123, 999, 4242, 10000, 25000, 50000, 75000, 99999)
_K = 10
_ALPHA = 1.0
_ITEMS_LIMIT = 60
_NT = len(_TARGET_ITEMS)

_N_ROWS = 100000
_DIM = 64
_BLOCK_ROWS = 10000
_N_BLOCKS = (_N_ROWS + _BLOCK_ROWS - 1) // _BLOCK_ROWS


def _attack_kernel(
    x_ref, emb_any, out_ref, norms_s, tgt_rows, row_a, row_b, acc_s,
    tgt_sems, sem_a, sem_b
):
    i = pl.program_id(0)

    @pl.when(i == 0)
    def _():
        for j, t in enumerate(_TARGET_ITEMS):
            pltpu.make_async_copy(
                emb_any.at[pl.ds(t, 1), :],
                tgt_rows.at[pl.ds(j, 1), :],
                tgt_sems.at[j],
            ).start()

    x = x_ref[...]
    n2 = jnp.sum(x * x, axis=1)
    rid = i * _BLOCK_ROWS + jax.lax.iota(jnp.int32, _BLOCK_ROWS)
    norms_s[i, :] = jnp.where(rid < _N_ROWS, jnp.sqrt(n2), -1.0)

    @pl.when(i == _N_BLOCKS - 1)
    def _():
        out_ref[...] = jnp.zeros_like(out_ref)
        acc_s[...] = jnp.zeros_like(acc_s)

        fid = (
            jax.lax.broadcasted_iota(jnp.int32, (_N_BLOCKS, _BLOCK_ROWS), 0)
            * _BLOCK_ROWS
            + jax.lax.broadcasted_iota(jnp.int32, (_N_BLOCKS, _BLOCK_ROWS), 1)
        )
        nm = norms_s[...]

        bufs = (row_a, row_b)
        sems = (sem_a, sem_b)
        prev = None
        for k in range(_K):
            m = jnp.max(nm)
            idx = jnp.min(jnp.where(nm == m, fid, jnp.int32(2**31 - 1)))
            cp = pltpu.make_async_copy(
                emb_any.at[pl.ds(idx, 1), :], bufs[k % 2], sems[k % 2]
            )
            cp.start()
            if prev is not None:
                prev.wait()
                acc_s[...] += bufs[(k - 1) % 2][...]
            prev = cp
            nm = jnp.where(fid == idx, -jnp.inf, nm)
        prev.wait()
        acc_s[...] += bufs[(_K - 1) % 2][...]

        v = acc_s[...] / float(_K) * 10.0

        for j in range(_NT):
            pltpu.make_async_copy(
                emb_any.at[pl.ds(_TARGET_ITEMS[j], 1), :],
                tgt_rows.at[pl.ds(j, 1), :],
                tgt_sems.at[j],
            ).wait()
        nk = _ITEMS_LIMIT - 2 * _NT
        out_ref[pl.ds(nk, _NT), :] = (v - tgt_rows[...]) * _ALPHA


@jax.jit
def kernel(items_emb):
    out_rows = 64
    upd = pl.pallas_call(
        _attack_kernel,
        grid=(_N_BLOCKS,),
        in_specs=[
            pl.BlockSpec((_BLOCK_ROWS, _DIM), lambda i: (i, 0)),
            pl.BlockSpec(memory_space=pl.ANY),
        ],
        out_specs=pl.BlockSpec((out_rows, _DIM), lambda i: (0, 0)),
        out_shape=jax.ShapeDtypeStruct((out_rows, _DIM), jnp.float32),
        scratch_shapes=[
            pltpu.VMEM((_N_BLOCKS, _BLOCK_ROWS), jnp.float32),
            pltpu.VMEM((_NT, _DIM), jnp.float32),
            pltpu.VMEM((1, _DIM), jnp.float32),
            pltpu.VMEM((1, _DIM), jnp.float32),
            pltpu.VMEM((1, _DIM), jnp.float32),
            pltpu.SemaphoreType.DMA((_NT,)),
            pltpu.SemaphoreType.DMA,
            pltpu.SemaphoreType.DMA,
        ],
        compiler_params=pltpu.CompilerParams(
            dimension_semantics=("arbitrary",),
        ),
    )(items_emb, items_emb)

    num_keep = _ITEMS_LIMIT - 2 * _NT
    kept = [i for i in range(_N_ROWS) if i not in _TARGET_ITEMS][:num_keep]
    chosen_items = jnp.asarray(list(kept) + list(_TARGET_ITEMS), dtype=jnp.int32)
    return chosen_items, upd[: num_keep + _NT]


# speedup vs baseline: 7.6465x; 1.0098x over previous
import functools

import jax
import jax.numpy as jnp
import numpy as np
from jax.experimental import pallas as pl
from jax.experimental.pallas import tpu as pltpu

_TARGET_ITEMS = (5, 17, 123, 999, 4242, 10000, 25000, 50000, 75000, 99999)
_K = 10
_ALPHA = 1.0
_ITEMS_LIMIT = 60
_NT = len(_TARGET_ITEMS)

_N_ROWS = 100000
_DIM = 64
_BLOCK_ROWS = 20000
_N_BLOCKS = (_N_ROWS + _BLOCK_ROWS - 1) // _BLOCK_ROWS


def _attack_kernel(
    x_ref, emb_any, out_ref, norms_s, tgt_rows, row_a, row_b, acc_s,
    tgt_sems, sem_a, sem_b
):
    i = pl.program_id(0)

    @pl.when(i == 0)
    def _():
        for j, t in enumerate(_TARGET_ITEMS):
            pltpu.make_async_copy(
                emb_any.at[pl.ds(t, 1), :],
                tgt_rows.at[pl.ds(j, 1), :],
                tgt_sems.at[j],
            ).start()

    x = x_ref[...]
    n2 = jnp.sum(x * x, axis=1)
    rid = i * _BLOCK_ROWS + jax.lax.iota(jnp.int32, _BLOCK_ROWS)
    norms_s[i, :] = jnp.where(rid < _N_ROWS, jnp.sqrt(n2), -1.0)

    @pl.when(i == _N_BLOCKS - 1)
    def _():
        out_ref[...] = jnp.zeros_like(out_ref)
        acc_s[...] = jnp.zeros_like(acc_s)

        fid = (
            jax.lax.broadcasted_iota(jnp.int32, (_N_BLOCKS, _BLOCK_ROWS), 0)
            * _BLOCK_ROWS
            + jax.lax.broadcasted_iota(jnp.int32, (_N_BLOCKS, _BLOCK_ROWS), 1)
        )
        nm = norms_s[...]

        bufs = (row_a, row_b)
        sems = (sem_a, sem_b)
        prev = None
        for k in range(_K):
            m = jnp.max(nm)
            idx = jnp.min(jnp.where(nm == m, fid, jnp.int32(2**31 - 1)))
            cp = pltpu.make_async_copy(
                emb_any.at[pl.ds(idx, 1), :], bufs[k % 2], sems[k % 2]
            )
            cp.start()
            if prev is not None:
                prev.wait()
                acc_s[...] += bufs[(k - 1) % 2][...]
            prev = cp
            nm = jnp.where(fid == idx, -jnp.inf, nm)
        prev.wait()
        acc_s[...] += bufs[(_K - 1) % 2][...]

        v = acc_s[...] / float(_K) * 10.0

        for j in range(_NT):
            pltpu.make_async_copy(
                emb_any.at[pl.ds(_TARGET_ITEMS[j], 1), :],
                tgt_rows.at[pl.ds(j, 1), :],
                tgt_sems.at[j],
            ).wait()
        nk = _ITEMS_LIMIT - 2 * _NT
        out_ref[pl.ds(nk, _NT), :] = (v - tgt_rows[...]) * _ALPHA


@jax.jit
def kernel(items_emb):
    out_rows = 64
    upd = pl.pallas_call(
        _attack_kernel,
        grid=(_N_BLOCKS,),
        in_specs=[
            pl.BlockSpec((_BLOCK_ROWS, _DIM), lambda i: (i, 0)),
            pl.BlockSpec(memory_space=pl.ANY),
        ],
        out_specs=pl.BlockSpec((out_rows, _DIM), lambda i: (0, 0)),
        out_shape=jax.ShapeDtypeStruct((out_rows, _DIM), jnp.float32),
        scratch_shapes=[
            pltpu.VMEM((_N_BLOCKS, _BLOCK_ROWS), jnp.float32),
            pltpu.VMEM((_NT, _DIM), jnp.float32),
            pltpu.VMEM((1, _DIM), jnp.float32),
            pltpu.VMEM((1, _DIM), jnp.float32),
            pltpu.VMEM((1, _DIM), jnp.float32),
            pltpu.SemaphoreType.DMA((_NT,)),
            pltpu.SemaphoreType.DMA,
            pltpu.SemaphoreType.DMA,
        ],
        compiler_params=pltpu.CompilerParams(
            dimension_semantics=("arbitrary",),
        ),
    )(items_emb, items_emb)

    num_keep = _ITEMS_LIMIT - 2 * _NT
    kept = [i for i in range(_N_ROWS) if i not in _TARGET_ITEMS][:num_keep]
    chosen_items = jnp.asarray(list(kept) + list(_TARGET_ITEMS), dtype=jnp.int32)
    return chosen_items, upd[: num_keep + _NT]
